# baseline (speedup 1.0000x reference)
import jax
import jax.numpy as jnp
from jax.experimental import pallas as pl


def _copy3(x_ref, e_ref, u_ref, xo_ref, eo_ref, uo_ref):
    xo_ref[...] = x_ref[...]
    eo_ref[...] = e_ref[...]
    uo_ref[...] = u_ref[...]


def kernel(x, edge_index, edge_attr, u, batch):
    del edge_index, batch
    et = edge_attr.T
    grid = 1
    xb = x.shape[0] // grid
    eb = et.shape[1] // grid
    outs = pl.pallas_call(
        _copy3,
        grid=(grid,),
        in_specs=[
            pl.BlockSpec((xb, x.shape[1]), lambda i: (i, 0)),
            pl.BlockSpec((et.shape[0], eb), lambda i: (0, i)),
            pl.BlockSpec(u.shape, lambda i: (0, 0)),
        ],
        out_specs=[
            pl.BlockSpec((xb, x.shape[1]), lambda i: (i, 0)),
            pl.BlockSpec((et.shape[0], eb), lambda i: (0, i)),
            pl.BlockSpec(u.shape, lambda i: (0, 0)),
        ],
        out_shape=[
            jax.ShapeDtypeStruct(x.shape, x.dtype),
            jax.ShapeDtypeStruct(et.shape, et.dtype),
            jax.ShapeDtypeStruct(u.shape, u.dtype),
        ],
    )(x, et, u)
    return (outs[0], outs[1].T, outs[2])


# speedup vs baseline: 1.1558x; 1.1558x over previous
import jax
import jax.numpy as jnp
from jax.experimental import pallas as pl


def _copy3(x_ref, e_ref, u_ref, xo_ref, eo_ref, uo_ref):
    xo_ref[...] = x_ref[...]
    eo_ref[...] = e_ref[...]
    uo_ref[...] = u_ref[...]


def kernel(x, edge_index, edge_attr, u, batch):
    del edge_index, batch
    et = edge_attr.T
    grid = 2
    xb = x.shape[0] // grid
    eb = et.shape[1] // grid
    outs = pl.pallas_call(
        _copy3,
        grid=(grid,),
        in_specs=[
            pl.BlockSpec((xb, x.shape[1]), lambda i: (i, 0)),
            pl.BlockSpec((et.shape[0], eb), lambda i: (0, i)),
            pl.BlockSpec(u.shape, lambda i: (0, 0)),
        ],
        out_specs=[
            pl.BlockSpec((xb, x.shape[1]), lambda i: (i, 0)),
            pl.BlockSpec((et.shape[0], eb), lambda i: (0, i)),
            pl.BlockSpec(u.shape, lambda i: (0, 0)),
        ],
        out_shape=[
            jax.ShapeDtypeStruct(x.shape, x.dtype),
            jax.ShapeDtypeStruct(et.shape, et.dtype),
            jax.ShapeDtypeStruct(u.shape, u.dtype),
        ],
    )(x, et, u)
    return (outs[0], outs[1].T, outs[2])
